# baseline (device time: 30475 ns/iter reference)
import jax
import jax.numpy as jnp
from jax import lax
from jax.experimental import pallas as pl
from jax.experimental.pallas import tpu as pltpu

N_DEV = 16
M = 512
QA = M // 4
QB = QA // 4


def _mask_a(o: int) -> int:
    return (o & 1) | ((o >> 1) << 2)


def _mask_b(o: int) -> int:
    return ((o & 1) << 1) | ((o >> 1) << 3)


def kernel(A, B):
    m, k = A.shape
    _, n = B.shape

    def body(a_ref, b_ref, out_ref, acc_ref, rbufA, rbufB,
             rsA_send, rsA_recv, rsB_send, rsB_recv,
             agA_send, agA_recv, agB_send, agB_recv):
        my = lax.axis_index("i")

        def bit(i):
            return jnp.bitwise_and(jnp.right_shift(my, i), 1)

        qmA = bit(0) + 2 * bit(2)
        qmB = bit(1) + 2 * bit(3)
        startA = QA * qmA
        startB = startA + QB * qmB

        barrier_sem = pltpu.get_barrier_semaphore()
        for mask in [_mask_a(o) for o in (1, 2, 3)] + [_mask_b(o) for o in (1, 2, 3)]:
            pl.semaphore_signal(
                barrier_sem, inc=1,
                device_id=(jnp.bitwise_xor(my, mask),),
                device_id_type=pl.DeviceIdType.MESH,
            )
        all_rdmas = []

        rsA = {}
        for o in (3, 2, 1):
            q = jnp.bitwise_xor(qmA, o)
            acc_ref[pl.ds(QA * q, QA), :] = jnp.dot(
                a_ref[pl.ds(QA * q, QA), :], b_ref[...],
                preferred_element_type=jnp.float32,
            )
            if o == 3:
                pl.semaphore_wait(barrier_sem, 6)
            partner = jnp.bitwise_xor(my, _mask_a(o))
            for oc in range(4):
                chunk = jnp.bitwise_xor(qmB, oc)
                rdma = pltpu.make_async_remote_copy(
                    src_ref=acc_ref.at[pl.ds(QA * q + QB * chunk, QB)],
                    dst_ref=rbufA.at[o - 1].at[pl.ds(QB * chunk, QB)],
                    send_sem=rsA_send.at[4 * (o - 1) + oc],
                    recv_sem=rsA_recv.at[4 * (o - 1) + oc],
                    device_id=(partner,),
                    device_id_type=pl.DeviceIdType.MESH,
                )
                rdma.start()
                rsA[(o, oc)] = rdma
        all_rdmas.extend(rsA.values())
        acc_ref[pl.ds(startA, QA), :] = jnp.dot(
            a_ref[pl.ds(startA, QA), :], b_ref[...],
            preferred_element_type=jnp.float32,
        )

        rsB = {}
        for oc in range(4):
            boff = QB * jnp.bitwise_xor(qmB, oc)
            coff = startA + boff
            for o in (1, 2, 3):
                rsA[(o, oc)].wait_recv()
            acc_ref[pl.ds(coff, QB), :] = (
                acc_ref[pl.ds(coff, QB), :]
                + rbufA[0, pl.ds(boff, QB), :]
                + rbufA[1, pl.ds(boff, QB), :]
                + rbufA[2, pl.ds(boff, QB), :]
            )
            if oc > 0:
                rdma = pltpu.make_async_remote_copy(
                    src_ref=acc_ref.at[pl.ds(coff, QB)],
                    dst_ref=rbufB.at[oc - 1],
                    send_sem=rsB_send.at[oc - 1],
                    recv_sem=rsB_recv.at[oc - 1],
                    device_id=(jnp.bitwise_xor(my, _mask_b(oc)),),
                    device_id_type=pl.DeviceIdType.MESH,
                )
                rdma.start()
                rsB[oc] = rdma
        all_rdmas.extend(rsB.values())

        for oc in (1, 2, 3):
            rsB[oc].wait_recv()
        out_ref[pl.ds(startB, QB), :] = (
            acc_ref[pl.ds(startB, QB), :]
            + rbufB[0] + rbufB[1] + rbufB[2]
        )

        agB = {}
        for ob in (1, 2, 3):
            rdma = pltpu.make_async_remote_copy(
                src_ref=out_ref.at[pl.ds(startB, QB)],
                dst_ref=out_ref.at[pl.ds(startB, QB)],
                send_sem=agB_send.at[ob - 1],
                recv_sem=agB_recv.at[ob - 1],
                device_id=(jnp.bitwise_xor(my, _mask_b(ob)),),
                device_id_type=pl.DeviceIdType.MESH,
            )
            rdma.start()
            agB[ob] = rdma
        agA = {}
        for o in (1, 2, 3):
            rdma = pltpu.make_async_remote_copy(
                src_ref=out_ref.at[pl.ds(startB, QB)],
                dst_ref=out_ref.at[pl.ds(startB, QB)],
                send_sem=agA_send.at[4 * (o - 1)],
                recv_sem=agA_recv.at[4 * (o - 1)],
                device_id=(jnp.bitwise_xor(my, _mask_a(o)),),
                device_id_type=pl.DeviceIdType.MESH,
            )
            rdma.start()
            agA[(o, 0)] = rdma

        for ob in (1, 2, 3):
            agB[ob].wait_recv()
            soff = startA + QB * jnp.bitwise_xor(qmB, ob)
            for o in (1, 2, 3):
                rdma = pltpu.make_async_remote_copy(
                    src_ref=out_ref.at[pl.ds(soff, QB)],
                    dst_ref=out_ref.at[pl.ds(soff, QB)],
                    send_sem=agA_send.at[4 * (o - 1) + ob],
                    recv_sem=agA_recv.at[4 * (o - 1) + ob],
                    device_id=(jnp.bitwise_xor(my, _mask_a(o)),),
                    device_id_type=pl.DeviceIdType.MESH,
                )
                rdma.start()
                agA[(o, ob)] = rdma
        all_rdmas.extend(agB.values())
        all_rdmas.extend(agA.values())

        for o in (1, 2, 3):
            for ob in range(4):
                agA[(o, ob)].wait_recv()
        for rdma in all_rdmas:
            rdma.wait_send()

    return pl.pallas_call(
        body,
        out_shape=jax.ShapeDtypeStruct((m, n), jnp.float32),
        in_specs=[
            pl.BlockSpec(memory_space=pltpu.VMEM),
            pl.BlockSpec(memory_space=pltpu.VMEM),
        ],
        out_specs=pl.BlockSpec(memory_space=pltpu.VMEM),
        scratch_shapes=[
            pltpu.VMEM((m, n), jnp.float32),
            pltpu.VMEM((3, QA, n), jnp.float32),
            pltpu.VMEM((3, QB, n), jnp.float32),
            pltpu.SemaphoreType.DMA((12,)),
            pltpu.SemaphoreType.DMA((12,)),
            pltpu.SemaphoreType.DMA((3,)),
            pltpu.SemaphoreType.DMA((3,)),
            pltpu.SemaphoreType.DMA((12,)),
            pltpu.SemaphoreType.DMA((12,)),
            pltpu.SemaphoreType.DMA((3,)),
            pltpu.SemaphoreType.DMA((3,)),
        ],
        compiler_params=pltpu.CompilerParams(collective_id=0),
    )(A, B)


# device time: 28444 ns/iter; 1.0714x vs baseline; 1.0714x over previous
import jax
import jax.numpy as jnp
from jax import lax
from jax.experimental import pallas as pl
from jax.experimental.pallas import tpu as pltpu

N_DEV = 16
M = 512
QA = M // 4
QB = QA // 4


def _mask_a(o: int) -> int:
    return (o & 1) | ((o >> 1) << 2)


def _mask_b(o: int) -> int:
    return ((o & 1) << 1) | ((o >> 1) << 3)


def kernel(A, B):
    m, k = A.shape
    _, n = B.shape

    def body(a_ref, b_ref, out_ref, acc_ref, rbufA, rbufB,
             rsA_send, rsA_recv, rsB_send, rsB_recv,
             agA_send, agA_recv, agB_send, agB_recv):
        my = lax.axis_index("i")

        def bit(i):
            return jnp.bitwise_and(jnp.right_shift(my, i), 1)

        qmA = bit(0) + 2 * bit(2)
        qmB = bit(1) + 2 * bit(3)
        startA = QA * qmA
        startB = startA + QB * qmB

        barrier_sem = pltpu.get_barrier_semaphore()
        for mask in [_mask_a(o) for o in (1, 2, 3)] + [_mask_b(o) for o in (1, 2, 3)]:
            pl.semaphore_signal(
                barrier_sem, inc=1,
                device_id=(jnp.bitwise_xor(my, mask),),
                device_id_type=pl.DeviceIdType.MESH,
            )
        pl.semaphore_wait(barrier_sem, 6)

        all_rdmas = []

        rsA = {}
        for o in (3, 2, 1):
            q = jnp.bitwise_xor(qmA, o)
            acc_ref[pl.ds(QA * q, QA), :] = jnp.dot(
                a_ref[pl.ds(QA * q, QA), :], b_ref[...],
                preferred_element_type=jnp.float32,
            )
            partner = jnp.bitwise_xor(my, _mask_a(o))
            for oc in range(4):
                chunk = jnp.bitwise_xor(qmB, oc)
                rdma = pltpu.make_async_remote_copy(
                    src_ref=acc_ref.at[pl.ds(QA * q + QB * chunk, QB)],
                    dst_ref=rbufA.at[o - 1].at[pl.ds(QB * chunk, QB)],
                    send_sem=rsA_send.at[4 * (o - 1) + oc],
                    recv_sem=rsA_recv.at[4 * (o - 1) + oc],
                    device_id=(partner,),
                    device_id_type=pl.DeviceIdType.MESH,
                )
                rdma.start()
                rsA[(o, oc)] = rdma
        all_rdmas.extend(rsA.values())
        acc_ref[pl.ds(startA, QA), :] = jnp.dot(
            a_ref[pl.ds(startA, QA), :], b_ref[...],
            preferred_element_type=jnp.float32,
        )

        rsB = {}
        for oc in range(4):
            boff = QB * jnp.bitwise_xor(qmB, oc)
            coff = startA + boff
            for o in (1, 2, 3):
                rsA[(o, oc)].wait_recv()
            acc_ref[pl.ds(coff, QB), :] = (
                acc_ref[pl.ds(coff, QB), :]
                + rbufA[0, pl.ds(boff, QB), :]
                + rbufA[1, pl.ds(boff, QB), :]
                + rbufA[2, pl.ds(boff, QB), :]
            )
            if oc > 0:
                rdma = pltpu.make_async_remote_copy(
                    src_ref=acc_ref.at[pl.ds(coff, QB)],
                    dst_ref=rbufB.at[oc - 1],
                    send_sem=rsB_send.at[oc - 1],
                    recv_sem=rsB_recv.at[oc - 1],
                    device_id=(jnp.bitwise_xor(my, _mask_b(oc)),),
                    device_id_type=pl.DeviceIdType.MESH,
                )
                rdma.start()
                rsB[oc] = rdma
        all_rdmas.extend(rsB.values())

        for oc in (1, 2, 3):
            rsB[oc].wait_recv()
        out_ref[pl.ds(startB, QB), :] = (
            acc_ref[pl.ds(startB, QB), :]
            + rbufB[0] + rbufB[1] + rbufB[2]
        )

        agB = {}
        for ob in (1, 2, 3):
            rdma = pltpu.make_async_remote_copy(
                src_ref=out_ref.at[pl.ds(startB, QB)],
                dst_ref=out_ref.at[pl.ds(startB, QB)],
                send_sem=agB_send.at[ob - 1],
                recv_sem=agB_recv.at[ob - 1],
                device_id=(jnp.bitwise_xor(my, _mask_b(ob)),),
                device_id_type=pl.DeviceIdType.MESH,
            )
            rdma.start()
            agB[ob] = rdma
        agA = {}
        for o in (1, 2, 3):
            rdma = pltpu.make_async_remote_copy(
                src_ref=out_ref.at[pl.ds(startB, QB)],
                dst_ref=out_ref.at[pl.ds(startB, QB)],
                send_sem=agA_send.at[4 * (o - 1)],
                recv_sem=agA_recv.at[4 * (o - 1)],
                device_id=(jnp.bitwise_xor(my, _mask_a(o)),),
                device_id_type=pl.DeviceIdType.MESH,
            )
            rdma.start()
            agA[(o, 0)] = rdma

        for ob in (1, 2, 3):
            agB[ob].wait_recv()
            soff = startA + QB * jnp.bitwise_xor(qmB, ob)
            for o in (1, 2, 3):
                rdma = pltpu.make_async_remote_copy(
                    src_ref=out_ref.at[pl.ds(soff, QB)],
                    dst_ref=out_ref.at[pl.ds(soff, QB)],
                    send_sem=agA_send.at[4 * (o - 1) + ob],
                    recv_sem=agA_recv.at[4 * (o - 1) + ob],
                    device_id=(jnp.bitwise_xor(my, _mask_a(o)),),
                    device_id_type=pl.DeviceIdType.MESH,
                )
                rdma.start()
                agA[(o, ob)] = rdma
        all_rdmas.extend(agB.values())
        all_rdmas.extend(agA.values())

        for o in (1, 2, 3):
            for ob in range(4):
                agA[(o, ob)].wait_recv()
        for rdma in all_rdmas:
            rdma.wait_send()

    return pl.pallas_call(
        body,
        out_shape=jax.ShapeDtypeStruct((m, n), jnp.float32),
        in_specs=[
            pl.BlockSpec(memory_space=pltpu.VMEM),
            pl.BlockSpec(memory_space=pltpu.VMEM),
        ],
        out_specs=pl.BlockSpec(memory_space=pltpu.VMEM),
        scratch_shapes=[
            pltpu.VMEM((m, n), jnp.float32),
            pltpu.VMEM((3, QA, n), jnp.float32),
            pltpu.VMEM((3, QB, n), jnp.float32),
            pltpu.SemaphoreType.DMA((12,)),
            pltpu.SemaphoreType.DMA((12,)),
            pltpu.SemaphoreType.DMA((3,)),
            pltpu.SemaphoreType.DMA((3,)),
            pltpu.SemaphoreType.DMA((12,)),
            pltpu.SemaphoreType.DMA((12,)),
            pltpu.SemaphoreType.DMA((3,)),
            pltpu.SemaphoreType.DMA((3,)),
        ],
        compiler_params=pltpu.CompilerParams(collective_id=0),
    )(A, B)


# device time: 27342 ns/iter; 1.1146x vs baseline; 1.0403x over previous
import jax
import jax.numpy as jnp
from jax import lax
from jax.experimental import pallas as pl
from jax.experimental.pallas import tpu as pltpu

N_DEV = 16
M = 512
QA = M // 4
QB = QA // 4


def _mask_a(o: int) -> int:
    return o


def _mask_b(o: int) -> int:
    return o << 2


def kernel(A, B):
    m, k = A.shape
    _, n = B.shape

    def body(a_ref, b_ref, out_ref, acc_ref, rbufA, rbufB,
             rsA_send, rsA_recv, rsB_send, rsB_recv,
             agA_send, agA_recv, agB_send, agB_recv):
        my = lax.axis_index("i")

        def bit(i):
            return jnp.bitwise_and(jnp.right_shift(my, i), 1)

        qmA = bit(0) + 2 * bit(1)
        qmB = bit(2) + 2 * bit(3)
        startA = QA * qmA
        startB = startA + QB * qmB

        barrier_sem = pltpu.get_barrier_semaphore()
        for mask in [_mask_a(o) for o in (1, 2, 3)] + [_mask_b(o) for o in (1, 2, 3)]:
            pl.semaphore_signal(
                barrier_sem, inc=1,
                device_id=(jnp.bitwise_xor(my, mask),),
                device_id_type=pl.DeviceIdType.MESH,
            )
        pl.semaphore_wait(barrier_sem, 6)

        all_rdmas = []

        rsA = {}
        for o in (3, 2, 1):
            q = jnp.bitwise_xor(qmA, o)
            acc_ref[pl.ds(QA * q, QA), :] = jnp.dot(
                a_ref[pl.ds(QA * q, QA), :], b_ref[...],
                preferred_element_type=jnp.float32,
            )
            partner = jnp.bitwise_xor(my, _mask_a(o))
            for oc in range(4):
                chunk = jnp.bitwise_xor(qmB, oc)
                rdma = pltpu.make_async_remote_copy(
                    src_ref=acc_ref.at[pl.ds(QA * q + QB * chunk, QB)],
                    dst_ref=rbufA.at[o - 1].at[pl.ds(QB * chunk, QB)],
                    send_sem=rsA_send.at[4 * (o - 1) + oc],
                    recv_sem=rsA_recv.at[4 * (o - 1) + oc],
                    device_id=(partner,),
                    device_id_type=pl.DeviceIdType.MESH,
                )
                rdma.start()
                rsA[(o, oc)] = rdma
        all_rdmas.extend(rsA.values())
        acc_ref[pl.ds(startA, QA), :] = jnp.dot(
            a_ref[pl.ds(startA, QA), :], b_ref[...],
            preferred_element_type=jnp.float32,
        )

        rsB = {}
        for oc in range(4):
            boff = QB * jnp.bitwise_xor(qmB, oc)
            coff = startA + boff
            for o in (1, 2, 3):
                rsA[(o, oc)].wait_recv()
            acc_ref[pl.ds(coff, QB), :] = (
                acc_ref[pl.ds(coff, QB), :]
                + rbufA[0, pl.ds(boff, QB), :]
                + rbufA[1, pl.ds(boff, QB), :]
                + rbufA[2, pl.ds(boff, QB), :]
            )
            if oc > 0:
                rdma = pltpu.make_async_remote_copy(
                    src_ref=acc_ref.at[pl.ds(coff, QB)],
                    dst_ref=rbufB.at[oc - 1],
                    send_sem=rsB_send.at[oc - 1],
                    recv_sem=rsB_recv.at[oc - 1],
                    device_id=(jnp.bitwise_xor(my, _mask_b(oc)),),
                    device_id_type=pl.DeviceIdType.MESH,
                )
                rdma.start()
                rsB[oc] = rdma
        all_rdmas.extend(rsB.values())

        for oc in (1, 2, 3):
            rsB[oc].wait_recv()
        out_ref[pl.ds(startB, QB), :] = (
            acc_ref[pl.ds(startB, QB), :]
            + rbufB[0] + rbufB[1] + rbufB[2]
        )

        agB = {}
        for ob in (1, 2, 3):
            rdma = pltpu.make_async_remote_copy(
                src_ref=out_ref.at[pl.ds(startB, QB)],
                dst_ref=out_ref.at[pl.ds(startB, QB)],
                send_sem=agB_send.at[ob - 1],
                recv_sem=agB_recv.at[ob - 1],
                device_id=(jnp.bitwise_xor(my, _mask_b(ob)),),
                device_id_type=pl.DeviceIdType.MESH,
            )
            rdma.start()
            agB[ob] = rdma
        agA = {}
        for o in (1, 2, 3):
            rdma = pltpu.make_async_remote_copy(
                src_ref=out_ref.at[pl.ds(startB, QB)],
                dst_ref=out_ref.at[pl.ds(startB, QB)],
                send_sem=agA_send.at[4 * (o - 1)],
                recv_sem=agA_recv.at[4 * (o - 1)],
                device_id=(jnp.bitwise_xor(my, _mask_a(o)),),
                device_id_type=pl.DeviceIdType.MESH,
            )
            rdma.start()
            agA[(o, 0)] = rdma

        for ob in (1, 2, 3):
            agB[ob].wait_recv()
            soff = startA + QB * jnp.bitwise_xor(qmB, ob)
            for o in (1, 2, 3):
                rdma = pltpu.make_async_remote_copy(
                    src_ref=out_ref.at[pl.ds(soff, QB)],
                    dst_ref=out_ref.at[pl.ds(soff, QB)],
                    send_sem=agA_send.at[4 * (o - 1) + ob],
                    recv_sem=agA_recv.at[4 * (o - 1) + ob],
                    device_id=(jnp.bitwise_xor(my, _mask_a(o)),),
                    device_id_type=pl.DeviceIdType.MESH,
                )
                rdma.start()
                agA[(o, ob)] = rdma
        all_rdmas.extend(agB.values())
        all_rdmas.extend(agA.values())

        for o in (1, 2, 3):
            for ob in range(4):
                agA[(o, ob)].wait_recv()
        for rdma in all_rdmas:
            rdma.wait_send()

    return pl.pallas_call(
        body,
        out_shape=jax.ShapeDtypeStruct((m, n), jnp.float32),
        in_specs=[
            pl.BlockSpec(memory_space=pltpu.VMEM),
            pl.BlockSpec(memory_space=pltpu.VMEM),
        ],
        out_specs=pl.BlockSpec(memory_space=pltpu.VMEM),
        scratch_shapes=[
            pltpu.VMEM((m, n), jnp.float32),
            pltpu.VMEM((3, QA, n), jnp.float32),
            pltpu.VMEM((3, QB, n), jnp.float32),
            pltpu.SemaphoreType.DMA((12,)),
            pltpu.SemaphoreType.DMA((12,)),
            pltpu.SemaphoreType.DMA((3,)),
            pltpu.SemaphoreType.DMA((3,)),
            pltpu.SemaphoreType.DMA((12,)),
            pltpu.SemaphoreType.DMA((12,)),
            pltpu.SemaphoreType.DMA((3,)),
            pltpu.SemaphoreType.DMA((3,)),
        ],
        compiler_params=pltpu.CompilerParams(collective_id=0),
    )(A, B)


# device time: 25543 ns/iter; 1.1931x vs baseline; 1.0704x over previous
import jax
import jax.numpy as jnp
from jax import lax
from jax.experimental import pallas as pl
from jax.experimental.pallas import tpu as pltpu

N_DEV = 16
M = 512
QA = M // 4
QB = QA // 4


def _mask_a(o: int) -> int:
    return o


def _mask_b(o: int) -> int:
    return o << 2


def kernel(A, B):
    m, k = A.shape
    _, n = B.shape

    def body(a_ref, b_ref, out_ref, acc_ref, rbufA, rbufB,
             rsA_send, rsA_recv, rsB_send, rsB_recv,
             agA_send, agA_recv, agB_send, agB_recv):
        my = lax.axis_index("i")

        def bit(i):
            return jnp.bitwise_and(jnp.right_shift(my, i), 1)

        qmA = bit(0) + 2 * bit(1)
        qmB = bit(2) + 2 * bit(3)
        startA = QA * qmA
        startB = startA + QB * qmB

        barrier_sem = pltpu.get_barrier_semaphore()
        for mask in [_mask_a(o) for o in (1, 2, 3)] + [_mask_b(o) for o in (1, 2, 3)]:
            pl.semaphore_signal(
                barrier_sem, inc=1,
                device_id=(jnp.bitwise_xor(my, mask),),
                device_id_type=pl.DeviceIdType.MESH,
            )
        pl.semaphore_wait(barrier_sem, 6)

        all_rdmas = []

        rsA = {}
        for o in (3, 2, 1):
            q = jnp.bitwise_xor(qmA, o)
            acc_ref[pl.ds(QA * q, QA), :] = jnp.dot(
                a_ref[pl.ds(QA * q, QA), :], b_ref[...],
                preferred_element_type=jnp.float32,
            )
            partner = jnp.bitwise_xor(my, _mask_a(o))
            for oc in (3, 2, 1, 0):
                chunk = jnp.bitwise_xor(qmB, oc)
                rdma = pltpu.make_async_remote_copy(
                    src_ref=acc_ref.at[pl.ds(QA * q + QB * chunk, QB)],
                    dst_ref=rbufA.at[o - 1].at[pl.ds(QB * chunk, QB)],
                    send_sem=rsA_send.at[4 * (o - 1) + oc],
                    recv_sem=rsA_recv.at[4 * (o - 1) + oc],
                    device_id=(partner,),
                    device_id_type=pl.DeviceIdType.MESH,
                )
                rdma.start()
                rsA[(o, oc)] = rdma
        all_rdmas.extend(rsA.values())
        acc_ref[pl.ds(startA, QA), :] = jnp.dot(
            a_ref[pl.ds(startA, QA), :], b_ref[...],
            preferred_element_type=jnp.float32,
        )

        rsB = {}
        for oc in (3, 2, 1, 0):
            boff = QB * jnp.bitwise_xor(qmB, oc)
            coff = startA + boff
            for o in (1, 2, 3):
                rsA[(o, oc)].wait_recv()
            acc_ref[pl.ds(coff, QB), :] = (
                acc_ref[pl.ds(coff, QB), :]
                + rbufA[0, pl.ds(boff, QB), :]
                + rbufA[1, pl.ds(boff, QB), :]
                + rbufA[2, pl.ds(boff, QB), :]
            )
            if oc > 0:
                rdma = pltpu.make_async_remote_copy(
                    src_ref=acc_ref.at[pl.ds(coff, QB)],
                    dst_ref=rbufB.at[oc - 1],
                    send_sem=rsB_send.at[oc - 1],
                    recv_sem=rsB_recv.at[oc - 1],
                    device_id=(jnp.bitwise_xor(my, _mask_b(oc)),),
                    device_id_type=pl.DeviceIdType.MESH,
                )
                rdma.start()
                rsB[oc] = rdma
        all_rdmas.extend(rsB.values())

        for oc in (1, 2, 3):
            rsB[oc].wait_recv()
        out_ref[pl.ds(startB, QB), :] = (
            acc_ref[pl.ds(startB, QB), :]
            + rbufB[0] + rbufB[1] + rbufB[2]
        )

        agB = {}
        for ob in (1, 2, 3):
            rdma = pltpu.make_async_remote_copy(
                src_ref=out_ref.at[pl.ds(startB, QB)],
                dst_ref=out_ref.at[pl.ds(startB, QB)],
                send_sem=agB_send.at[ob - 1],
                recv_sem=agB_recv.at[ob - 1],
                device_id=(jnp.bitwise_xor(my, _mask_b(ob)),),
                device_id_type=pl.DeviceIdType.MESH,
            )
            rdma.start()
            agB[ob] = rdma
        agA = {}
        for o in (1, 2, 3):
            rdma = pltpu.make_async_remote_copy(
                src_ref=out_ref.at[pl.ds(startB, QB)],
                dst_ref=out_ref.at[pl.ds(startB, QB)],
                send_sem=agA_send.at[4 * (o - 1)],
                recv_sem=agA_recv.at[4 * (o - 1)],
                device_id=(jnp.bitwise_xor(my, _mask_a(o)),),
                device_id_type=pl.DeviceIdType.MESH,
            )
            rdma.start()
            agA[(o, 0)] = rdma

        for ob in (1, 2, 3):
            agB[ob].wait_recv()
            soff = startA + QB * jnp.bitwise_xor(qmB, ob)
            for o in (1, 2, 3):
                rdma = pltpu.make_async_remote_copy(
                    src_ref=out_ref.at[pl.ds(soff, QB)],
                    dst_ref=out_ref.at[pl.ds(soff, QB)],
                    send_sem=agA_send.at[4 * (o - 1) + ob],
                    recv_sem=agA_recv.at[4 * (o - 1) + ob],
                    device_id=(jnp.bitwise_xor(my, _mask_a(o)),),
                    device_id_type=pl.DeviceIdType.MESH,
                )
                rdma.start()
                agA[(o, ob)] = rdma
        all_rdmas.extend(agB.values())
        all_rdmas.extend(agA.values())

        for o in (1, 2, 3):
            for ob in range(4):
                agA[(o, ob)].wait_recv()
        for rdma in all_rdmas:
            rdma.wait_send()

    return pl.pallas_call(
        body,
        out_shape=jax.ShapeDtypeStruct((m, n), jnp.float32),
        in_specs=[
            pl.BlockSpec(memory_space=pltpu.VMEM),
            pl.BlockSpec(memory_space=pltpu.VMEM),
        ],
        out_specs=pl.BlockSpec(memory_space=pltpu.VMEM),
        scratch_shapes=[
            pltpu.VMEM((m, n), jnp.float32),
            pltpu.VMEM((3, QA, n), jnp.float32),
            pltpu.VMEM((3, QB, n), jnp.float32),
            pltpu.SemaphoreType.DMA((12,)),
            pltpu.SemaphoreType.DMA((12,)),
            pltpu.SemaphoreType.DMA((3,)),
            pltpu.SemaphoreType.DMA((3,)),
            pltpu.SemaphoreType.DMA((12,)),
            pltpu.SemaphoreType.DMA((12,)),
            pltpu.SemaphoreType.DMA((3,)),
            pltpu.SemaphoreType.DMA((3,)),
        ],
        compiler_params=pltpu.CompilerParams(collective_id=0),
    )(A, B)
